# Initial kernel scaffold; baseline (speedup 1.0000x reference)
import jax
import jax.numpy as jnp
from jax.experimental import pallas as pl


def kernel(router_logits):
    raise NotImplementedError("write your pallas kernel here")


# speedup vs baseline: 4.9043x; 4.9043x over previous
import functools

import jax
import jax.numpy as jnp
from jax import lax
from jax.experimental import pallas as pl
from jax.experimental.pallas import tpu as pltpu
from jax.experimental.pallas import tpu_sc as plsc

TOP_K = 8
EPS = 0.2
NUM_TOKENS = 32768
NUM_EXPERTS = 64

NUM_CORES = 2
NUM_SUBCORES = 16
LANES = 16
NUM_WORKERS = NUM_CORES * NUM_SUBCORES
ROWS_PER_W = NUM_TOKENS // NUM_WORKERS
GROUPS = ROWS_PER_W // LANES
CHUNK_WORDS = ROWS_PER_W * NUM_EXPERTS
OUT_WORDS = ROWS_PER_W * TOP_K

_T_POS = 1.0 - 2.0 * EPS
_T_NEG = 1.0 / (1.0 - 2.0 * EPS)


def _sc_body(logits_hbm, idx_hbm, val_hbm, chunk, lt, et, idxs, vals):
    wid = lax.axis_index("s") * NUM_CORES + lax.axis_index("c")
    pltpu.sync_copy(logits_hbm.at[pl.ds(wid * CHUNK_WORDS, CHUNK_WORDS)], chunk)

    lanes = lax.iota(jnp.int32, 16)

    def group_body(g, carry):
        rows = g * LANES + lanes
        row_base = rows * NUM_EXPERTS
        m = jnp.full((16,), -jnp.inf, jnp.float32)
        idxv = jnp.zeros((16,), jnp.int32)
        for j in range(NUM_EXPERTS):
            v = plsc.load_gather(chunk, [row_base + j])
            lt[pl.ds(j * LANES, LANES)] = v
            upd = v > m
            m = jnp.where(upd, v, m)
            idxv = jnp.where(upd, jnp.int32(j), idxv)
        m0 = m
        for j in range(NUM_EXPERTS):
            s = pl.ds(j * LANES, LANES)
            et[s] = jnp.exp(lt[s] - m0)
        out_base = rows * TOP_K
        for i in range(TOP_K):
            t = jnp.where(m >= 0, _T_POS * m, _T_NEG * m)
            num = jnp.exp(m - m0)
            plsc.store_scatter(idxs, [out_base + i], idxv)
            plsc.store_scatter(lt, [idxv * LANES + lanes],
                               jnp.full((16,), -jnp.inf, jnp.float32))
            acc = jnp.zeros((16,), jnp.float32)
            nm = jnp.full((16,), -jnp.inf, jnp.float32)
            nidx = jnp.zeros((16,), jnp.int32)
            for j in range(NUM_EXPERTS):
                s = pl.ds(j * LANES, LANES)
                lv = lt[s]
                ev = et[s]
                acc = acc + jnp.where(lv >= t, ev, jnp.float32(0.0))
                upd = lv > nm
                nm = jnp.where(upd, lv, nm)
                nidx = jnp.where(upd, jnp.int32(j), nidx)
            val = num / (acc + num)
            plsc.store_scatter(vals, [out_base + i], val)
            m = nm
            idxv = nidx
        return carry

    lax.fori_loop(0, GROUPS, group_body, jnp.int32(0))

    pltpu.sync_copy(idxs, idx_hbm.at[pl.ds(wid * OUT_WORDS, OUT_WORDS)])
    pltpu.sync_copy(vals, val_hbm.at[pl.ds(wid * OUT_WORDS, OUT_WORDS)])


_sc_call = functools.partial(
    pl.kernel,
    out_type=(
        jax.ShapeDtypeStruct((NUM_TOKENS * TOP_K,), jnp.int32),
        jax.ShapeDtypeStruct((NUM_TOKENS * TOP_K,), jnp.float32),
    ),
    mesh=plsc.VectorSubcoreMesh(core_axis_name="c", subcore_axis_name="s"),
    compiler_params=pltpu.CompilerParams(needs_layout_passes=False),
    scratch_types=[
        pltpu.VMEM((CHUNK_WORDS,), jnp.float32),
        pltpu.VMEM((NUM_EXPERTS * LANES,), jnp.float32),
        pltpu.VMEM((NUM_EXPERTS * LANES,), jnp.float32),
        pltpu.VMEM((OUT_WORDS,), jnp.int32),
        pltpu.VMEM((OUT_WORDS,), jnp.float32),
    ],
)(_sc_body)


@jax.jit
def kernel(router_logits):
    flat = jnp.reshape(router_logits.astype(jnp.float32), (-1,))
    idx_flat, val_flat = _sc_call(flat)
    return (jnp.reshape(idx_flat, (NUM_TOKENS, TOP_K)),
            jnp.reshape(val_flat, (NUM_TOKENS, TOP_K)))
